# baseline (speedup 1.0000x reference)
import jax
import jax.numpy as jnp
from jax.experimental import pallas as pl
from jax.experimental.pallas import tpu as pltpu

_N_LABELS = 4
_BLOCK_ROWS = 8000


def _ggnn_body(e_ref, x_ref, w_ref, o_ref):
    x = x_ref[...]
    eb = e_ref[0]
    b, n = eb.shape
    ecol = eb[:, :, None]
    x3 = x.astype(jnp.bfloat16).reshape(b, n, x.shape[1])
    zero = jnp.zeros((), jnp.bfloat16)
    x4 = jnp.concatenate(
        [jnp.where(ecol == i, x3, zero) for i in range(_N_LABELS)], axis=-1,
    ).reshape(x.shape[0], _N_LABELS * x.shape[1])
    o_ref[...] = jax.lax.dot_general(
        x4, w_ref[...],
        dimension_numbers=(((1,), (0,)), ((), ())),
        preferred_element_type=jnp.float32,
    )


def kernel(h_v, h_w, e_vw, edge_matrix):
    del h_v
    nb, nn, nin = h_w.shape
    nout = edge_matrix.shape[1]
    rows = nb * nn
    br = _BLOCK_ROWS
    grid = (rows // br,)
    em_t = jnp.transpose(edge_matrix, (0, 2, 1)).astype(jnp.bfloat16)
    em_s = jnp.reshape(em_t, (_N_LABELS * nin, nout))
    x2 = jnp.reshape(h_w, (rows, nin))
    e2 = jnp.reshape(e_vw, (rows // br, br // nn, nn))
    out = pl.pallas_call(
        _ggnn_body,
        grid=grid,
        in_specs=[
            pl.BlockSpec((1, br // nn, nn), lambda g: (g, 0, 0)),
            pl.BlockSpec((br, nin), lambda g: (g, 0)),
            pl.BlockSpec((_N_LABELS * nin, nout), lambda g: (0, 0)),
        ],
        out_specs=pl.BlockSpec((br, nout), lambda g: (g, 0)),
        out_shape=jax.ShapeDtypeStruct((rows, nout), jnp.float32),
        compiler_params=pltpu.CompilerParams(
            dimension_semantics=("arbitrary",),
        ),
    )(e2, x2, em_s)
    return jnp.reshape(out, (nb, nn, nout))


# speedup vs baseline: 1.1169x; 1.1169x over previous
import jax
import jax.numpy as jnp
from jax.experimental import pallas as pl
from jax.experimental.pallas import tpu as pltpu

_N_LABELS = 4
_BLOCK_ROWS = 20000


def _ggnn_body(e_ref, x_ref, w_ref, o_ref):
    x = x_ref[...]
    eb = e_ref[0]
    b, n = eb.shape
    ecol = eb[:, :, None]
    x3 = x.astype(jnp.bfloat16).reshape(b, n, x.shape[1])
    zero = jnp.zeros((), jnp.bfloat16)
    x4 = jnp.concatenate(
        [jnp.where(ecol == i, x3, zero) for i in range(_N_LABELS)], axis=-1,
    ).reshape(x.shape[0], _N_LABELS * x.shape[1])
    o_ref[...] = jax.lax.dot_general(
        x4, w_ref[...],
        dimension_numbers=(((1,), (0,)), ((), ())),
        preferred_element_type=jnp.float32,
    )


def kernel(h_v, h_w, e_vw, edge_matrix):
    del h_v
    nb, nn, nin = h_w.shape
    nout = edge_matrix.shape[1]
    rows = nb * nn
    br = _BLOCK_ROWS
    grid = (rows // br,)
    em_t = jnp.transpose(edge_matrix, (0, 2, 1)).astype(jnp.bfloat16)
    em_s = jnp.reshape(em_t, (_N_LABELS * nin, nout))
    x2 = jnp.reshape(h_w, (rows, nin))
    e2 = jnp.reshape(e_vw, (rows // br, br // nn, nn))
    out = pl.pallas_call(
        _ggnn_body,
        grid=grid,
        in_specs=[
            pl.BlockSpec((1, br // nn, nn), lambda g: (g, 0, 0)),
            pl.BlockSpec((br, nin), lambda g: (g, 0)),
            pl.BlockSpec((_N_LABELS * nin, nout), lambda g: (0, 0)),
        ],
        out_specs=pl.BlockSpec((br, nout), lambda g: (g, 0)),
        out_shape=jax.ShapeDtypeStruct((rows, nout), jnp.float32),
        compiler_params=pltpu.CompilerParams(
            dimension_semantics=("arbitrary",),
        ),
    )(e2, x2, em_s)
    return jnp.reshape(out, (nb, nn, nout))


# speedup vs baseline: 1.2412x; 1.1112x over previous
import jax
import jax.numpy as jnp
from jax.experimental import pallas as pl
from jax.experimental.pallas import tpu as pltpu

_N_LABELS = 4
_CHUNK = 8000
_DEPTH = 4


def _compute_chunk(xb_slot, eb_slot, w_ref, ob_slot):
    x = xb_slot[...]
    eb = eb_slot[...]
    b, n = eb.shape
    ecol = eb[:, :, None]
    x3 = x.astype(jnp.bfloat16).reshape(b, n, x.shape[1])
    zero = jnp.zeros((), jnp.bfloat16)
    x4 = jnp.concatenate(
        [jnp.where(ecol == i, x3, zero) for i in range(_N_LABELS)], axis=-1,
    ).reshape(x.shape[0], _N_LABELS * x.shape[1])
    ob_slot[...] = jax.lax.dot_general(
        x4, w_ref[...],
        dimension_numbers=(((1,), (0,)), ((), ())),
        preferred_element_type=jnp.float32,
    )


def _pipeline_body(e_hbm, x_hbm, w_ref, o_hbm,
                   xbuf, ebuf, obuf, x_sem, e_sem, o_sem):
    rows = x_hbm.shape[0]
    n_chunks = rows // _CHUNK
    erows = _CHUNK // 32

    def x_copy(i, slot):
        return pltpu.make_async_copy(
            x_hbm.at[pl.ds(i * _CHUNK, _CHUNK), :], xbuf.at[slot],
            x_sem.at[slot])

    def e_copy(i, slot):
        return pltpu.make_async_copy(
            e_hbm.at[pl.ds(i * erows, erows), :], ebuf.at[slot],
            e_sem.at[slot])

    def o_copy(i, slot):
        return pltpu.make_async_copy(
            obuf.at[slot], o_hbm.at[pl.ds(i * _CHUNK, _CHUNK), :],
            o_sem.at[slot])

    for j in range(_DEPTH - 1):
        x_copy(j, j).start()
        e_copy(j, j).start()

    def step(i, carry):
        slot = jax.lax.rem(i, _DEPTH)

        @pl.when(i + _DEPTH - 1 < n_chunks)
        def _prefetch():
            nslot = jax.lax.rem(i + _DEPTH - 1, _DEPTH)
            x_copy(i + _DEPTH - 1, nslot).start()
            e_copy(i + _DEPTH - 1, nslot).start()

        x_copy(i, slot).wait()
        e_copy(i, slot).wait()

        @pl.when(i >= _DEPTH)
        def _drain():
            o_copy(i - _DEPTH, slot).wait()

        _compute_chunk(xbuf.at[slot], ebuf.at[slot], w_ref, obuf.at[slot])
        o_copy(i, slot).start()
        return carry

    jax.lax.fori_loop(0, n_chunks, step, 0)

    for j in range(_DEPTH):
        i = n_chunks - _DEPTH + j
        o_copy(i, i % _DEPTH).wait()


def kernel(h_v, h_w, e_vw, edge_matrix):
    del h_v
    nb, nn, nin = h_w.shape
    nout = edge_matrix.shape[1]
    rows = nb * nn
    em_t = jnp.transpose(edge_matrix, (0, 2, 1)).astype(jnp.bfloat16)
    em_s = jnp.reshape(em_t, (_N_LABELS * nin, nout))
    x2 = jnp.reshape(h_w, (rows, nin))
    e2 = jnp.reshape(e_vw, (rows // nn, nn))
    out = pl.pallas_call(
        _pipeline_body,
        in_specs=[
            pl.BlockSpec(memory_space=pl.ANY),
            pl.BlockSpec(memory_space=pl.ANY),
            pl.BlockSpec(memory_space=pltpu.MemorySpace.VMEM),
        ],
        out_specs=pl.BlockSpec(memory_space=pl.ANY),
        out_shape=jax.ShapeDtypeStruct((rows, nout), jnp.float32),
        scratch_shapes=[
            pltpu.VMEM((_DEPTH, _CHUNK, nin), jnp.float32),
            pltpu.VMEM((_DEPTH, _CHUNK // nn, nn), jnp.int32),
            pltpu.VMEM((_DEPTH, _CHUNK, nout), jnp.float32),
            pltpu.SemaphoreType.DMA((_DEPTH,)),
            pltpu.SemaphoreType.DMA((_DEPTH,)),
            pltpu.SemaphoreType.DMA((_DEPTH,)),
        ],
    )(e2, x2, em_s)
    return jnp.reshape(out, (nb, nn, nout))
